# baseline (speedup 1.0000x reference)
import jax
import jax.numpy as jnp
from jax.experimental import pallas as pl

_BE = 10000


def _add_kernel(e0, e1, r0, r1, out_e, out_r):
    out_e[...] = e0[...] + e1[...]

    @pl.when(pl.program_id(0) == 0)
    def _():
        out_r[...] = r0[...] + r1[...]


def kernel(inputs, ent_embeds_0, rel_embeds_0, ent_embeds_1, rel_embeds_1):
    n_ent, d_ent = ent_embeds_0.shape
    n_rel, d_rel = rel_embeds_0.shape
    grid = (pl.cdiv(n_ent, _BE),)
    ent_spec = pl.BlockSpec((_BE, d_ent), lambda i: (i, 0))
    rel_spec = pl.BlockSpec((n_rel, d_rel), lambda i: (0, 0))
    out_ent, out_rel = pl.pallas_call(
        _add_kernel,
        grid=grid,
        in_specs=[ent_spec, ent_spec, rel_spec, rel_spec],
        out_specs=[ent_spec, rel_spec],
        out_shape=[
            jax.ShapeDtypeStruct((n_ent, d_ent), ent_embeds_0.dtype),
            jax.ShapeDtypeStruct((n_rel, d_rel), rel_embeds_0.dtype),
        ],
    )(ent_embeds_0, ent_embeds_1, rel_embeds_0, rel_embeds_1)
    return (out_ent, out_rel)


# speedup vs baseline: 6.1211x; 6.1211x over previous
import jax
import jax.numpy as jnp
from jax.experimental import pallas as pl
from jax.experimental.pallas import tpu as pltpu

_BC = 8192


def _add_kernel(e0, e1, r0, r1, out_e, out_r):
    out_e[...] = e0[...] + e1[...]

    @pl.when(pl.program_id(0) == 0)
    def _():
        out_r[...] = r0[...] + r1[...]


def kernel(inputs, ent_embeds_0, rel_embeds_0, ent_embeds_1, rel_embeds_1):
    n_ent, d_ent = ent_embeds_0.shape
    n_rel, d_rel = rel_embeds_0.shape
    e0t, e1t = ent_embeds_0.T, ent_embeds_1.T
    r0t, r1t = rel_embeds_0.T, rel_embeds_1.T
    grid = (pl.cdiv(n_ent, _BC),)
    ent_spec = pl.BlockSpec((d_ent, _BC), lambda i: (0, i))
    rel_spec = pl.BlockSpec((d_rel, n_rel), lambda i: (0, 0))
    out_et, out_rt = pl.pallas_call(
        _add_kernel,
        grid=grid,
        in_specs=[ent_spec, ent_spec, rel_spec, rel_spec],
        out_specs=[ent_spec, rel_spec],
        out_shape=[
            jax.ShapeDtypeStruct((d_ent, n_ent), ent_embeds_0.dtype),
            jax.ShapeDtypeStruct((d_rel, n_rel), rel_embeds_0.dtype),
        ],
        compiler_params=pltpu.CompilerParams(
            dimension_semantics=("arbitrary",),
        ),
    )(e0t, e1t, r0t, r1t)
    return (out_et.T, out_rt.T)


# speedup vs baseline: 6.1466x; 1.0042x over previous
import jax
import jax.numpy as jnp
from jax.experimental import pallas as pl
from jax.experimental.pallas import tpu as pltpu

_BC = 8192


def _add_kernel(e0, e1, r0, r1, out_e, out_r):
    out_e[...] = e0[...] + e1[...]

    @pl.when(pl.program_id(0) == 0)
    def _():
        out_r[...] = r0[...] + r1[...]


def kernel(inputs, ent_embeds_0, rel_embeds_0, ent_embeds_1, rel_embeds_1):
    n_ent, d_ent = ent_embeds_0.shape
    n_rel, d_rel = rel_embeds_0.shape
    e0t, e1t = ent_embeds_0.T, ent_embeds_1.T
    r0t, r1t = rel_embeds_0.T, rel_embeds_1.T
    grid = (pl.cdiv(n_ent, _BC),)
    ent_spec = pl.BlockSpec((d_ent, _BC), lambda i: (0, i))
    rel_spec = pl.BlockSpec((d_rel, n_rel), lambda i: (0, 0))
    out_et, out_rt = pl.pallas_call(
        _add_kernel,
        grid=grid,
        in_specs=[ent_spec, ent_spec, rel_spec, rel_spec],
        out_specs=[ent_spec, rel_spec],
        out_shape=[
            jax.ShapeDtypeStruct((d_ent, n_ent), ent_embeds_0.dtype),
            jax.ShapeDtypeStruct((d_rel, n_rel), rel_embeds_0.dtype),
        ],
        compiler_params=pltpu.CompilerParams(
            dimension_semantics=("parallel",),
        ),
    )(e0t, e1t, r0t, r1t)
    return (out_et.T, out_rt.T)


# speedup vs baseline: 6.3463x; 1.0325x over previous
import jax
import jax.numpy as jnp
from jax.experimental import pallas as pl
from jax.experimental.pallas import tpu as pltpu

_BC = 16384


def _add_kernel(e0, e1, r0, r1, out_e, out_r):
    out_e[...] = e0[...] + e1[...]

    @pl.when(pl.program_id(0) == 0)
    def _():
        out_r[...] = r0[...] + r1[...]


def kernel(inputs, ent_embeds_0, rel_embeds_0, ent_embeds_1, rel_embeds_1):
    n_ent, d_ent = ent_embeds_0.shape
    n_rel, d_rel = rel_embeds_0.shape
    e0t, e1t = ent_embeds_0.T, ent_embeds_1.T
    r0t, r1t = rel_embeds_0.T, rel_embeds_1.T
    grid = (pl.cdiv(n_ent, _BC),)
    ent_spec = pl.BlockSpec((d_ent, _BC), lambda i: (0, i))
    rel_spec = pl.BlockSpec((d_rel, n_rel), lambda i: (0, 0))
    out_et, out_rt = pl.pallas_call(
        _add_kernel,
        grid=grid,
        in_specs=[ent_spec, ent_spec, rel_spec, rel_spec],
        out_specs=[ent_spec, rel_spec],
        out_shape=[
            jax.ShapeDtypeStruct((d_ent, n_ent), ent_embeds_0.dtype),
            jax.ShapeDtypeStruct((d_rel, n_rel), rel_embeds_0.dtype),
        ],
        compiler_params=pltpu.CompilerParams(
            dimension_semantics=("parallel",),
        ),
    )(e0t, e1t, r0t, r1t)
    return (out_et.T, out_rt.T)
